# baseline (speedup 1.0000x reference)
import functools

import jax
import jax.numpy as jnp
from jax import lax
from jax.experimental import pallas as pl
from jax.experimental.pallas import tpu as pltpu
from jax.experimental.pallas import tpu_sc as plsc

B = 4096
L = 200
V = 1000000
D = 16
H = 50
O = 3
NTOK = B * L
NW = 32
TOK_PER_W = NTOK // NW
CHUNK = 2560
NCHUNK = TOK_PER_W // CHUNK
NPACK = NTOK // 8


def _sc_gather(table, idx):
  mesh = plsc.VectorSubcoreMesh(core_axis_name="c", subcore_axis_name="s")

  @functools.partial(
      pl.kernel,
      out_type=jax.ShapeDtypeStruct((NTOK, D), jnp.float32),
      mesh=mesh,
      compiler_params=pltpu.CompilerParams(use_tc_tiling_on_sc=False),
      scratch_types=[
          pltpu.VMEM((CHUNK,), jnp.int32),
          pltpu.VMEM((CHUNK, D), jnp.float32),
          pltpu.SemaphoreType.DMA,
      ],
  )
  def k(table_hbm, idx_hbm, out_hbm, idx_v, rows_v, sem):
    wid = lax.axis_index("s") * 2 + lax.axis_index("c")
    base = wid * TOK_PER_W

    def body(i, carry):
      off = base + i * CHUNK
      pltpu.sync_copy(idx_hbm.at[pl.ds(off, CHUNK)], idx_v)
      pltpu.async_copy(table_hbm.at[idx_v], rows_v, sem).wait()
      pltpu.sync_copy(rows_v, out_hbm.at[pl.ds(off, CHUNK)])
      return carry

    lax.fori_loop(0, NCHUNK, body, 0)

  return k(table, idx)


def _tc_mlp_packed(emb_p, W1, b1, W2, b2):
  BT = 6400
  BB = BT // L
  BTP = BT // 8
  grid = NTOK // BT

  w1bd = jnp.kron(jnp.eye(8, dtype=jnp.float32)[::-1], W1)
  b1bd = jnp.tile(b1, 8)
  w2bd = jnp.kron(jnp.eye(8, dtype=jnp.float32), W2)
  b2bd = jnp.tile(b2, 8)
  sbd = jnp.kron(jnp.eye(8, dtype=jnp.float32),
                 jnp.ones((O, O), jnp.float32))

  def body(ep_ref, w1_ref, b1_ref, w2_ref, b2_ref, s_ref, out_ref):
    ep = ep_ref[...]
    hp = jnp.dot(ep, w1_ref[...], preferred_element_type=jnp.float32)
    hp = jnp.maximum(hp + b1_ref[...], 0.0)
    lp = jnp.dot(hp, w2_ref[...], preferred_element_type=jnp.float32)
    lp = lp + b2_ref[...]
    m = jnp.max(lp, axis=-1, keepdims=True)
    el = jnp.exp(lp - m)
    ssum = jnp.dot(el, s_ref[...], preferred_element_type=jnp.float32)
    ls = lp - m - jnp.log(ssum)
    ls = jnp.pad(ls, ((0, 0), (0, 104)))
    l3 = lax.broadcast_in_dim(ls, (BTP, 8, 128), (0, 2))
    l3 = pltpu.roll(l3, 107, 2, stride=3, stride_axis=1)
    out_ref[...] = l3.reshape(BT, 128)[:, :O].reshape(BB, L, O)

  return pl.pallas_call(
      body,
      grid=(grid,),
      in_specs=[
          pl.BlockSpec((BTP, 128), lambda i: (i, 0)),
          pl.BlockSpec((128, 8 * H), lambda i: (0, 0)),
          pl.BlockSpec((8 * H,), lambda i: (0,)),
          pl.BlockSpec((8 * H, 8 * O), lambda i: (0, 0)),
          pl.BlockSpec((8 * O,), lambda i: (0,)),
          pl.BlockSpec((8 * O, 8 * O), lambda i: (0, 0)),
      ],
      out_specs=pl.BlockSpec((BB, L, O), lambda i: (i, 0, 0)),
      out_shape=jax.ShapeDtypeStruct((B, L, O), jnp.float32),
  )(emb_p, w1bd, b1bd, w2bd, b2bd, sbd)


def kernel(x, table, W1, b1, W2, b2):
  idx = x.reshape(NTOK).astype(jnp.int32)
  tbl_t = lax.optimization_barrier(lax.transpose(table, (1, 0)))
  tbl_lin = lax.reshape(lax.transpose(tbl_t, (1, 0)), (V * D,))
  tbl = lax.reshape(tbl_lin, (V, D))
  emb = _sc_gather(tbl, idx)
  out = _tc_mlp_packed(emb.reshape(NPACK, 128), W1, b1, W2, b2)
  return out


# speedup vs baseline: 1.0085x; 1.0085x over previous
import functools

import jax
import jax.numpy as jnp
from jax import lax
from jax.experimental import pallas as pl
from jax.experimental.pallas import tpu as pltpu
from jax.experimental.pallas import tpu_sc as plsc

B = 4096
L = 200
V = 1000000
D = 16
H = 50
O = 3
NTOK = B * L
NW = 32
TOK_PER_W = NTOK // NW
CHUNK = 2560
NCHUNK = TOK_PER_W // CHUNK
NPACK = NTOK // 8


def _sc_gather(table, idx):
  mesh = plsc.VectorSubcoreMesh(core_axis_name="c", subcore_axis_name="s")

  @functools.partial(
      pl.kernel,
      out_type=jax.ShapeDtypeStruct((NTOK, D), jnp.float32),
      mesh=mesh,
      compiler_params=pltpu.CompilerParams(use_tc_tiling_on_sc=False),
      scratch_types=[
          pltpu.VMEM((CHUNK,), jnp.int32),
          pltpu.VMEM((CHUNK,), jnp.int32),
          pltpu.VMEM((CHUNK, D), jnp.float32),
          pltpu.VMEM((CHUNK, D), jnp.float32),
          pltpu.SemaphoreType.DMA,
          pltpu.SemaphoreType.DMA,
          pltpu.SemaphoreType.DMA,
          pltpu.SemaphoreType.DMA,
      ],
  )
  def k(table_hbm, idx_hbm, out_hbm, idx_v0, idx_v1, rows_v0, rows_v1,
        g0, g1, w0, w1):
    wid = lax.axis_index("s") * 2 + lax.axis_index("c")
    base = wid * TOK_PER_W
    idx_v = (idx_v0, idx_v1)
    rows_v = (rows_v0, rows_v1)
    gsem = (g0, g1)
    wsem = (w0, w1)

    writes = [None, None]
    for i in range(NCHUNK):
      b = i % 2
      off = base + i * CHUNK
      if writes[b] is not None:
        writes[b].wait()
        writes[b] = None
      pltpu.sync_copy(idx_hbm.at[pl.ds(off, CHUNK)], idx_v[b])
      pltpu.async_copy(table_hbm.at[idx_v[b]], rows_v[b], gsem[b]).wait()
      writes[b] = pltpu.async_copy(rows_v[b],
                                   out_hbm.at[pl.ds(off, CHUNK)], wsem[b])
    for wr in writes:
      if wr is not None:
        wr.wait()

  return k(table, idx)


def _tc_mlp_packed(emb_p, W1, b1, W2, b2):
  BT = 6400
  BB = BT // L
  BTP = BT // 8
  grid = NTOK // BT

  w1bd = jnp.kron(jnp.eye(8, dtype=jnp.float32)[::-1], W1)
  b1bd = jnp.tile(b1, 8)
  w2bd = jnp.kron(jnp.eye(8, dtype=jnp.float32), W2)
  b2bd = jnp.tile(b2, 8)
  sbd = jnp.kron(jnp.eye(8, dtype=jnp.float32),
                 jnp.ones((O, O), jnp.float32))

  def body(ep_ref, w1_ref, b1_ref, w2_ref, b2_ref, s_ref, out_ref):
    ep = ep_ref[...]
    hp = jnp.dot(ep, w1_ref[...], preferred_element_type=jnp.float32)
    hp = jnp.maximum(hp + b1_ref[...], 0.0)
    lp = jnp.dot(hp, w2_ref[...], preferred_element_type=jnp.float32)
    lp = lp + b2_ref[...]
    m = jnp.max(lp, axis=-1, keepdims=True)
    el = jnp.exp(lp - m)
    ssum = jnp.dot(el, s_ref[...], preferred_element_type=jnp.float32)
    ls = lp - m - jnp.log(ssum)
    ls = jnp.pad(ls, ((0, 0), (0, 104)))
    l3 = lax.broadcast_in_dim(ls, (BTP, 8, 128), (0, 2))
    l3 = pltpu.roll(l3, 107, 2, stride=3, stride_axis=1)
    out_ref[...] = l3.reshape(BT, 128)[:, :O].reshape(BB, L, O)

  return pl.pallas_call(
      body,
      grid=(grid,),
      in_specs=[
          pl.BlockSpec((BTP, 128), lambda i: (i, 0)),
          pl.BlockSpec((128, 8 * H), lambda i: (0, 0)),
          pl.BlockSpec((8 * H,), lambda i: (0,)),
          pl.BlockSpec((8 * H, 8 * O), lambda i: (0, 0)),
          pl.BlockSpec((8 * O,), lambda i: (0,)),
          pl.BlockSpec((8 * O, 8 * O), lambda i: (0, 0)),
      ],
      out_specs=pl.BlockSpec((BB, L, O), lambda i: (i, 0, 0)),
      out_shape=jax.ShapeDtypeStruct((B, L, O), jnp.float32),
  )(emb_p, w1bd, b1bd, w2bd, b2bd, sbd)


def kernel(x, table, W1, b1, W2, b2):
  idx = x.reshape(NTOK).astype(jnp.int32)
  emb = _sc_gather(table, idx)
  return _tc_mlp_packed(emb.reshape(NPACK, 128), W1, b1, W2, b2)
